# Initial kernel scaffold; baseline (speedup 1.0000x reference)
import jax
import jax.numpy as jnp
from jax.experimental import pallas as pl


def kernel(feat, label, centers):
    raise NotImplementedError("write your pallas kernel here")


# speedup vs baseline: 6.1386x; 6.1386x over previous
import jax
import jax.numpy as jnp
from jax.experimental import pallas as pl
from jax.experimental.pallas import tpu as pltpu


def _body(feat_ref, label_ref, centers_ref, out_ref):
    f = feat_ref[...]
    lab = label_ref[...]
    c = centers_ref[...]
    c0 = c[0:1, :]
    c1 = c[1:2, :]
    c2 = c[2:3, :]
    is0 = lab == 0
    is1 = lab == 1
    is2 = lab == 2
    cb = jnp.where(is0, c0, jnp.where(is1, c1, c2))
    cb1 = jnp.where(is0, c1, c0)
    cb2 = jnp.where(is2, c1, c2)
    main = jnp.sum((f - cb) ** 2)
    d1 = jnp.sum((f - cb1) ** 2)
    d2 = jnp.sum((f - cb2) ** 2)
    b = f.shape[0]
    out_ref[0, 0] = main * (1.0 + 1.0 / (d1 + d2)) * (0.5 / b)


def kernel(feat, label, centers):
    b = feat.shape[0]
    lab2d = label.astype(jnp.int32).reshape(b, 1)
    out = pl.pallas_call(
        _body,
        out_shape=jax.ShapeDtypeStruct((1, 1), jnp.float32),
        in_specs=[
            pl.BlockSpec(memory_space=pltpu.VMEM),
            pl.BlockSpec(memory_space=pltpu.VMEM),
            pl.BlockSpec(memory_space=pltpu.VMEM),
        ],
        out_specs=pl.BlockSpec(memory_space=pltpu.SMEM),
    )(feat, lab2d, centers)
    return out.reshape(1)
